# Initial kernel scaffold; baseline (speedup 1.0000x reference)
import jax
import jax.numpy as jnp
from jax.experimental import pallas as pl


def kernel(x, category, numeric, emb, cat_table, W1, b1, W2, b2):
    raise NotImplementedError("write your pallas kernel here")


# speedup vs baseline: 3.5589x; 3.5589x over previous
import functools

import jax
import jax.numpy as jnp
from jax import lax
from jax.experimental import pallas as pl
from jax.experimental.pallas import tpu as pltpu
from jax.experimental.pallas import tpu_sc as plsc

NUM_CORES = 2
NUM_SUBCORES = 16
NW = NUM_CORES * NUM_SUBCORES
LANES = 16


def _sc_gather_pool(xi, cat_t, emb, cat_table):
  B = xi.shape[0]
  D = emb.shape[1]
  n_cat = cat_t.shape[0]
  bpw = B // NW
  assert B % (8 * NW) == 0

  mesh = plsc.VectorSubcoreMesh(core_axis_name="c", subcore_axis_name="s")

  @functools.partial(
      pl.kernel,
      out_type=(
          jax.ShapeDtypeStruct((B, D), jnp.float32),
          jax.ShapeDtypeStruct((B, D), jnp.float32),
      ),
      mesh=mesh,
      compiler_params=pltpu.CompilerParams(use_tc_tiling_on_sc=False),
      scratch_types=[
          pltpu.VMEM((bpw,), jnp.int32),
          pltpu.VMEM((bpw, D), jnp.float32),
          pltpu.VMEM((bpw, D), jnp.float32),
          pltpu.SemaphoreType.DMA,
      ],
  )
  def k(xi_hbm, catt_hbm, emb_hbm, ctab_hbm, uout_hbm, cout_hbm,
        idx_v, tmp_v, acc_v, sem):
    wid = lax.axis_index("s") * NUM_CORES + lax.axis_index("c")
    base = wid * bpw

    pltpu.sync_copy(xi_hbm.at[pl.ds(base, bpw)], idx_v)
    pltpu.async_copy(emb_hbm.at[idx_v], tmp_v, sem).wait()
    pltpu.sync_copy(tmp_v, uout_hbm.at[pl.ds(base, bpw)])

    pltpu.sync_copy(catt_hbm.at[0, pl.ds(base, bpw)], idx_v)
    pltpu.async_copy(ctab_hbm.at[idx_v], acc_v, sem).wait()

    for j in range(1, n_cat):
      pltpu.sync_copy(catt_hbm.at[j, pl.ds(base, bpw)], idx_v)
      pltpu.async_copy(ctab_hbm.at[idx_v], tmp_v, sem).wait()

      def body(i, _):
        for c in range(D // LANES):
          v = tmp_v[i, pl.ds(c * LANES, LANES)]
          plsc.addupdate(acc_v.at[i, pl.ds(c * LANES, LANES)], v)
        return 0

      lax.fori_loop(0, bpw, body, 0)

    pltpu.sync_copy(acc_v, cout_hbm.at[pl.ds(base, bpw)])

  return k(xi, cat_t, emb, cat_table)


def _tc_mlp(u, cp, numz, w1u, w1c, w1n, b1, w2, b2):
  B, D = u.shape
  H = w1u.shape[1]
  NP = numz.shape[1]
  BLK = 2048
  grid = (B // BLK,)

  def body(u_ref, c_ref, n_ref, w1u_ref, w1c_ref, w1n_ref, b1_ref, w2_ref,
           b2_ref, o_ref):
    h = jnp.dot(u_ref[...], w1u_ref[...], preferred_element_type=jnp.float32)
    h = h + jnp.dot(c_ref[...], w1c_ref[...],
                    preferred_element_type=jnp.float32)
    h = h + jnp.dot(n_ref[...], w1n_ref[...],
                    preferred_element_type=jnp.float32)
    h = jnp.maximum(h + b1_ref[...], 0.0)
    o_ref[...] = (jnp.dot(h, w2_ref[...], preferred_element_type=jnp.float32)
                  + b2_ref[0, 0])

  return pl.pallas_call(
      body,
      grid=grid,
      in_specs=[
          pl.BlockSpec((BLK, D), lambda i: (i, 0)),
          pl.BlockSpec((BLK, D), lambda i: (i, 0)),
          pl.BlockSpec((BLK, NP), lambda i: (i, 0)),
          pl.BlockSpec((D, H), lambda i: (0, 0)),
          pl.BlockSpec((D, H), lambda i: (0, 0)),
          pl.BlockSpec((NP, H), lambda i: (0, 0)),
          pl.BlockSpec((1, H), lambda i: (0, 0)),
          pl.BlockSpec((H, 1), lambda i: (0, 0)),
          pl.BlockSpec(memory_space=pltpu.SMEM),
      ],
      out_specs=pl.BlockSpec((BLK, 1), lambda i: (i, 0)),
      out_shape=jax.ShapeDtypeStruct((B, 1), jnp.float32),
  )(u, cp, numz, w1u, w1c, w1n, b1, w2, b2)


def kernel(x, category, numeric, emb, cat_table, W1, b1, W2, b2):
  B = x.shape[0]
  D = emb.shape[1]
  n_num = numeric.shape[1]

  xi = x[:, 0].astype(jnp.int32)
  cat_t = category.T.astype(jnp.int32)

  user_emb, cat_pooled = _sc_gather_pool(xi, cat_t, emb, cat_table)

  np_pad = 16
  numz = jnp.pad(numeric, ((0, 0), (0, np_pad - n_num)))
  w1u = W1[:D]
  w1c = W1[D:2 * D]
  w1n = jnp.pad(W1[2 * D:], ((0, np_pad - n_num), (0, 0)))
  b1r = b1.reshape(1, -1)
  b2r = b2.reshape(1, 1)

  return _tc_mlp(user_emb, cat_pooled, numz, w1u, w1c, w1n, b1r, W2, b2r)
